# baseline (speedup 1.0000x reference)
import jax
import jax.numpy as jnp
from jax import lax
from jax.experimental import pallas as pl
from jax.experimental.pallas import tpu as pltpu
from jax.experimental.pallas import tpu_sc as plsc

B = 16384
D = 32
L = 16
NC = 2
NS = 16
NW = NC * NS
B_PER_W = B // NW
CB = 128
N_CHUNKS = B_PER_W // CB


def _sc_body(hidx_hbm, tidx_hbm, ridx_hbm,
             ent_base, ent_trans, rc1, rw1, rc2, rw2,
             out_part, out_flags,
             idxh_v, idxt_v, idxr_v,
             hb_v, tb_v, ht_v, tt_v, c1_v, w1_v, c2_v, w2_v,
             part_v, flag_v, sem):
  wid = lax.axis_index("s") * NC + lax.axis_index("c")
  lane = lax.iota(jnp.int32, L)

  fl1 = jnp.zeros((L,), jnp.int32)
  fl2 = jnp.zeros((L,), jnp.int32)

  for k in range(N_CHUNKS):
    base = wid * B_PER_W + k * CB
    pltpu.sync_copy(hidx_hbm.at[pl.ds(base, CB)], idxh_v)
    pltpu.sync_copy(tidx_hbm.at[pl.ds(base, CB)], idxt_v)
    pltpu.sync_copy(ridx_hbm.at[pl.ds(base, CB)], idxr_v)
    cps = [
        pltpu.async_copy(ent_base.at[idxh_v], hb_v, sem),
        pltpu.async_copy(ent_base.at[idxt_v], tb_v, sem),
        pltpu.async_copy(ent_trans.at[idxh_v], ht_v, sem),
        pltpu.async_copy(ent_trans.at[idxt_v], tt_v, sem),
        pltpu.async_copy(rc1.at[idxr_v], c1_v, sem),
        pltpu.async_copy(rw1.at[idxr_v], w1_v, sem),
        pltpu.async_copy(rc2.at[idxr_v], c2_v, sem),
        pltpu.async_copy(rw2.at[idxr_v], w2_v, sem),
    ]
    for cp in cps:
      cp.wait()

    def group(g, fl):
      f1, f2 = fl
      rows = g * L + lane
      vi1 = jnp.zeros((L,), jnp.float32)
      vo1 = jnp.zeros((L,), jnp.float32)
      vi2 = jnp.zeros((L,), jnp.float32)
      vo2 = jnp.zeros((L,), jnp.float32)
      for d in range(D):
        dd = jnp.full((L,), d, jnp.int32)
        e = (plsc.load_gather(hb_v, [rows, dd])
             + plsc.load_gather(tt_v, [rows, dd]))
        c = plsc.load_gather(c1_v, [rows, dd])
        w = jnp.abs(plsc.load_gather(w1_v, [rows, dd])) + 1.0
        rw = 1.0 / w
        hw = 0.5 * (w - 1.0)
        kk = hw * (w - rw)
        a = jnp.abs(e - c)
        di = a * rw
        do = a * w - kk
        vi1 = vi1 + di * di
        vo1 = vo1 + do * do
        f1 = jnp.where(a > hw, 1, f1)
        e = (plsc.load_gather(tb_v, [rows, dd])
             + plsc.load_gather(ht_v, [rows, dd]))
        c = plsc.load_gather(c2_v, [rows, dd])
        w = jnp.abs(plsc.load_gather(w2_v, [rows, dd])) + 1.0
        rw = 1.0 / w
        hw = 0.5 * (w - 1.0)
        kk = hw * (w - rw)
        a = jnp.abs(e - c)
        di = a * rw
        do = a * w - kk
        vi2 = vi2 + di * di
        vo2 = vo2 + do * do
        f2 = jnp.where(a > hw, 1, f2)
      off = k * CB + g * L
      part_v[0, pl.ds(off, L)] = vi1
      part_v[1, pl.ds(off, L)] = vo1
      part_v[2, pl.ds(off, L)] = vi2
      part_v[3, pl.ds(off, L)] = vo2
      return (f1, f2)

    fl1, fl2 = lax.fori_loop(0, CB // L, group, (fl1, fl2))

  pltpu.sync_copy(part_v, out_part.at[wid])
  flag_v[:] = jnp.bitwise_or(fl1, jnp.left_shift(fl2, 1))
  pltpu.sync_copy(flag_v, out_flags.at[wid])


@jax.jit
def kernel(sample, ent_base, ent_trans, rel_c1, rel_w1, rel_c2, rel_w2):
  h_idx = sample[:, 0].astype(jnp.int32)
  r_idx = sample[:, 1].astype(jnp.int32)
  t_idx = sample[:, 2].astype(jnp.int32)

  mesh = plsc.VectorSubcoreMesh(core_axis_name="c", subcore_axis_name="s")
  call = pl.kernel(
      _sc_body,
      out_type=[
          jax.ShapeDtypeStruct((NW, 4, B_PER_W), jnp.float32),
          jax.ShapeDtypeStruct((NW, L), jnp.int32),
      ],
      mesh=mesh,
      compiler_params=pltpu.CompilerParams(needs_layout_passes=False,
                                           use_tc_tiling_on_sc=False),
      scratch_types=[
          pltpu.VMEM((CB,), jnp.int32),
          pltpu.VMEM((CB,), jnp.int32),
          pltpu.VMEM((CB,), jnp.int32),
          pltpu.VMEM((CB, D), jnp.float32),
          pltpu.VMEM((CB, D), jnp.float32),
          pltpu.VMEM((CB, D), jnp.float32),
          pltpu.VMEM((CB, D), jnp.float32),
          pltpu.VMEM((CB, D), jnp.float32),
          pltpu.VMEM((CB, D), jnp.float32),
          pltpu.VMEM((CB, D), jnp.float32),
          pltpu.VMEM((CB, D), jnp.float32),
          pltpu.VMEM((4, B_PER_W), jnp.float32),
          pltpu.VMEM((L,), jnp.int32),
          pltpu.SemaphoreType.DMA,
      ],
  )
  partials, flags = call(h_idx, t_idx, r_idx, ent_base, ent_trans,
                         rel_c1, rel_w1, rel_c2, rel_w2)

  p = jnp.transpose(partials, (1, 0, 2)).reshape(4, B)
  out1 = jnp.any(jnp.bitwise_and(flags, 1) != 0)
  out2 = jnp.any(jnp.bitwise_and(flags, 2) != 0)
  s1 = jnp.sqrt(jnp.where(out1, p[1], p[0]))
  s2 = jnp.sqrt(jnp.where(out2, p[3], p[2]))
  return s1 + s2
